# baseline (speedup 1.0000x reference)
import functools

import jax
import jax.numpy as jnp
from jax import lax
from jax.experimental import pallas as pl
from jax.experimental.pallas import tpu as pltpu
from jax.experimental.pallas import tpu_sc as plsc

_INFO = plsc.get_sparse_core_info()
_NC = _INFO.num_cores
_NS = _INFO.num_subcores
_L = _INFO.num_lanes
_NW = _NC * _NS

_N = 65536
_W = 512
_BPW = _N // _NW
_CH = 64
_NCHUNK = _BPW // _CH

_mesh = plsc.VectorSubcoreMesh(core_axis_name="c", subcore_axis_name="s")


@functools.partial(
    pl.kernel,
    mesh=_mesh,
    out_type=jax.ShapeDtypeStruct((_N,), jnp.float32),
    scratch_types=[
        pltpu.VMEM((_BPW,), jnp.int32),
        pltpu.VMEM((_BPW,), jnp.int32),
        pltpu.VMEM((_NCHUNK, _CH), jnp.int32),
        pltpu.VMEM((_BPW,), jnp.float32),
        pltpu.SemaphoreType.DMA,
        pltpu.SemaphoreType.DMA,
    ],
)
def _gather2d_sc(xs_hbm, ys_hbm, table_hbm, out_hbm, xs_v, ys_v, idx_v,
                 val_v, sem, sem_in):
    wid = lax.axis_index("s") * _NC + lax.axis_index("c")
    base = wid * _BPW
    _REST = _BPW - _CH
    c0 = pltpu.async_copy(xs_hbm.at[pl.ds(base, _CH)],
                          xs_v.at[pl.ds(0, _CH)], sem_in)
    c1 = pltpu.async_copy(ys_hbm.at[pl.ds(base, _CH)],
                          ys_v.at[pl.ds(0, _CH)], sem_in)
    c2 = pltpu.async_copy(xs_hbm.at[pl.ds(base + _CH, _REST)],
                          xs_v.at[pl.ds(_CH, _REST)], sem_in)
    c3 = pltpu.async_copy(ys_hbm.at[pl.ds(base + _CH, _REST)],
                          ys_v.at[pl.ds(_CH, _REST)], sem_in)

    def chunk(j, _):
        for k in range(_CH // _L):
            x = xs_v[pl.ds(j * _CH + k * _L, _L)]
            y = ys_v[pl.ds(j * _CH + k * _L, _L)]
            idx_v[j, pl.ds(k * _L, _L)] = y * _W + x
        pltpu.async_copy(table_hbm.at[idx_v.at[j]],
                         val_v.at[pl.ds(j * _CH, _CH)], sem)
        return 0

    c0.wait()
    c1.wait()
    chunk(0, 0)
    c2.wait()
    c3.wait()
    lax.fori_loop(1, _NCHUNK, chunk, 0)
    pltpu.make_async_copy(out_hbm.at[pl.ds(base, _BPW)], val_v, sem).wait()
    pltpu.sync_copy(val_v, out_hbm.at[pl.ds(base, _BPW)])


def kernel(kpts, im):
    kt = kpts.T
    out = _gather2d_sc(kt[0], kt[1], im[0, 0].reshape(-1))
    return out.reshape(1, 1, _N)


# speedup vs baseline: 1.0085x; 1.0085x over previous
import functools

import jax
import jax.numpy as jnp
from jax import lax
from jax.experimental import pallas as pl
from jax.experimental.pallas import tpu as pltpu
from jax.experimental.pallas import tpu_sc as plsc

_INFO = plsc.get_sparse_core_info()
_NC = _INFO.num_cores
_NS = _INFO.num_subcores
_L = _INFO.num_lanes
_NW = _NC * _NS

_N = 65536
_W = 512
_BPW = _N // _NW
_CH = 128
_NCHUNK = _BPW // _CH

_mesh = plsc.VectorSubcoreMesh(core_axis_name="c", subcore_axis_name="s")


@functools.partial(
    pl.kernel,
    mesh=_mesh,
    out_type=jax.ShapeDtypeStruct((_N,), jnp.float32),
    scratch_types=[
        pltpu.VMEM((_BPW,), jnp.int32),
        pltpu.VMEM((_BPW,), jnp.int32),
        pltpu.VMEM((_NCHUNK, _CH), jnp.int32),
        pltpu.VMEM((_BPW,), jnp.float32),
        pltpu.SemaphoreType.DMA,
        pltpu.SemaphoreType.DMA,
    ],
)
def _gather2d_sc(xs_hbm, ys_hbm, table_hbm, out_hbm, xs_v, ys_v, idx_v,
                 val_v, sem, sem_in):
    wid = lax.axis_index("s") * _NC + lax.axis_index("c")
    base = wid * _BPW
    _REST = _BPW - _CH
    c0 = pltpu.async_copy(xs_hbm.at[pl.ds(base, _CH)],
                          xs_v.at[pl.ds(0, _CH)], sem_in)
    c1 = pltpu.async_copy(ys_hbm.at[pl.ds(base, _CH)],
                          ys_v.at[pl.ds(0, _CH)], sem_in)
    c2 = pltpu.async_copy(xs_hbm.at[pl.ds(base + _CH, _REST)],
                          xs_v.at[pl.ds(_CH, _REST)], sem_in)
    c3 = pltpu.async_copy(ys_hbm.at[pl.ds(base + _CH, _REST)],
                          ys_v.at[pl.ds(_CH, _REST)], sem_in)

    def chunk(j, _):
        for k in range(_CH // _L):
            x = xs_v[pl.ds(j * _CH + k * _L, _L)]
            y = ys_v[pl.ds(j * _CH + k * _L, _L)]
            idx_v[j, pl.ds(k * _L, _L)] = y * _W + x
        pltpu.async_copy(table_hbm.at[idx_v.at[j]],
                         val_v.at[pl.ds(j * _CH, _CH)], sem)
        return 0

    c0.wait()
    c1.wait()
    chunk(0, 0)
    c2.wait()
    c3.wait()
    lax.fori_loop(1, _NCHUNK, chunk, 0)
    pltpu.make_async_copy(out_hbm.at[pl.ds(base, _BPW)], val_v, sem).wait()
    pltpu.sync_copy(val_v, out_hbm.at[pl.ds(base, _BPW)])


def kernel(kpts, im):
    kt = kpts.T
    out = _gather2d_sc(kt[0], kt[1], im[0, 0].reshape(-1))
    return out.reshape(1, 1, _N)
